# Initial kernel scaffold; baseline (speedup 1.0000x reference)
import jax
import jax.numpy as jnp
from jax.experimental import pallas as pl


def kernel(x, edge_index, edge_type, rel_emb, weight_neighbor, loop_weight, evolve_loop_weight):
    raise NotImplementedError("write your pallas kernel here")


# speedup vs baseline: 5.3288x; 5.3288x over previous
import functools

import jax
import jax.numpy as jnp
from jax import lax
from jax.experimental import pallas as pl
from jax.experimental.pallas import tpu as pltpu
from jax.experimental.pallas import tpu_sc as plsc

C = 128
NTILES = 32
SUB = 16


def _sc_pass(xh, relh, src4, dst4, et4, n_pad, chunks, with_deg):
    Dh = xh.shape[1]
    rows_d = n_pad // SUB
    zfull = rows_d // C
    ztail = rows_d - zfull * C

    mesh = plsc.VectorSubcoreMesh(core_axis_name="c", subcore_axis_name="s")

    out_type = [jax.ShapeDtypeStruct((2, n_pad, Dh), jnp.float32)]
    scratch = [
        pltpu.VMEM((chunks, C), jnp.int32),
        pltpu.VMEM((chunks, C), jnp.int32),
        pltpu.VMEM((chunks, C), jnp.int32),
        pltpu.VMEM((C, Dh), jnp.float32),
        pltpu.VMEM((C, Dh), jnp.float32),
        pltpu.VMEM_SHARED((n_pad, Dh), jnp.float32),
        pltpu.SemaphoreType.DMA,
        pltpu.SemaphoreType.DMA,
    ]
    if with_deg:
        out_type.append(jax.ShapeDtypeStruct((2, n_pad, SUB), jnp.float32))
        scratch += [
            pltpu.VMEM((C, SUB), jnp.float32),
            pltpu.VMEM((C, SUB), jnp.float32),
            pltpu.VMEM_SHARED((n_pad, SUB), jnp.float32),
        ]

    @functools.partial(
        pl.kernel, out_type=out_type, mesh=mesh, scratch_types=scratch,
        compiler_params=pltpu.CompilerParams(use_tc_tiling_on_sc=False))
    def k(x_hbm, rel_hbm, src_hbm, dst_hbm, et_hbm, *rest):
        if with_deg:
            (out_s, out_d, src_v, dst_v, et_v, gbuf, rbuf, s_acc,
             sem0, sem1, obuf, dz, d_acc) = rest
        else:
            (out_s, src_v, dst_v, et_v, gbuf, rbuf, s_acc,
             sem0, sem1) = rest
        cid = lax.axis_index("c")
        sid = lax.axis_index("s")

        zero16 = jnp.zeros((SUB,), jnp.float32)
        one0 = jnp.where(lax.iota(jnp.int32, SUB) == 0, 1.0, 0.0)

        def init_row(i, _):
            for j in range(Dh // SUB):
                gbuf[i, pl.ds(j * SUB, SUB)] = zero16
            if with_deg:
                obuf[i, pl.ds(0, SUB)] = one0
                dz[i, pl.ds(0, SUB)] = zero16
            return 0

        lax.fori_loop(0, C, init_row, 0)

        base = sid * rows_d
        for kk in range(zfull):
            pltpu.sync_copy(gbuf, s_acc.at[pl.ds(base + kk * C, C)])
            if with_deg:
                pltpu.sync_copy(dz, d_acc.at[pl.ds(base + kk * C, C)])
        if ztail:
            pltpu.sync_copy(gbuf.at[pl.ds(0, ztail)],
                            s_acc.at[pl.ds(base + zfull * C, ztail)])
            if with_deg:
                pltpu.sync_copy(dz.at[pl.ds(0, ztail)],
                                d_acc.at[pl.ds(base + zfull * C, ztail)])
        plsc.subcore_barrier()

        pltpu.sync_copy(src_hbm.at[cid, sid], src_v)
        pltpu.sync_copy(dst_hbm.at[cid, sid], dst_v)
        pltpu.sync_copy(et_hbm.at[cid, sid], et_v)

        def chunk(j, _):
            g1 = pltpu.async_copy(x_hbm.at[src_v.at[j]], gbuf, sem0)
            g2 = pltpu.async_copy(rel_hbm.at[et_v.at[j]], rbuf, sem1)
            g1.wait()
            g2.wait()
            pltpu.sync_copy(gbuf, s_acc.at[dst_v.at[j]], add=True)
            pltpu.sync_copy(rbuf, s_acc.at[dst_v.at[j]], add=True)
            if with_deg:
                pltpu.sync_copy(obuf, d_acc.at[dst_v.at[j]], add=True)
            return 0

        lax.fori_loop(0, chunks, chunk, 0)
        plsc.subcore_barrier()

        pltpu.sync_copy(s_acc.at[pl.ds(base, rows_d)],
                        out_s.at[cid, pl.ds(base, rows_d)])
        if with_deg:
            pltpu.sync_copy(d_acc.at[pl.ds(base, rows_d)],
                            out_d.at[cid, pl.ds(base, rows_d)])

    return k(xh, relh, src4, dst4, et4)


def _tc_combine(x, sa, sb, dg, wn, wl, we, blk):
    n, D = x.shape
    Dh = D // 2

    def body(x_ref, sa0, sa1, sb0, sb1, d0, d1, wn_ref, wl_ref, we_ref,
             o_ref):
        d = d0[0] + d1[0]
        deg = jnp.sum(d, axis=1, keepdims=True)
        has = deg > 0.0
        norm = jnp.where(has, 1.0 / jnp.maximum(deg, 1.0), 0.0)
        slo = sa0[0] + sa1[0]
        shi = sb0[0] + sb1[0]
        wnm = wn_ref[...]
        agg = (jnp.dot(slo, wnm[:Dh], preferred_element_type=jnp.float32)
               + jnp.dot(shi, wnm[Dh:], preferred_element_type=jnp.float32))
        xb = x_ref[...]
        lm = jnp.where(
            has,
            jnp.dot(xb, wl_ref[...], preferred_element_type=jnp.float32),
            jnp.dot(xb, we_ref[...], preferred_element_type=jnp.float32),
        )
        o_ref[...] = jnp.maximum(agg * norm + lm, 0.0)

    grid = (n // blk,)
    row_spec = pl.BlockSpec((blk, D), lambda i: (i, 0))
    s0_spec = pl.BlockSpec((1, blk, Dh), lambda i: (0, i, 0))
    s1_spec = pl.BlockSpec((1, blk, Dh), lambda i: (1, i, 0))
    d0_spec = pl.BlockSpec((1, blk, SUB), lambda i: (0, i, 0))
    d1_spec = pl.BlockSpec((1, blk, SUB), lambda i: (1, i, 0))
    w_spec = pl.BlockSpec((D, D), lambda i: (0, 0))
    return pl.pallas_call(
        body,
        grid=grid,
        in_specs=[row_spec, s0_spec, s1_spec, s0_spec, s1_spec,
                  d0_spec, d1_spec, w_spec, w_spec, w_spec],
        out_specs=row_spec,
        out_shape=jax.ShapeDtypeStruct((n, D), jnp.float32),
    )(x, sa, sa, sb, sb, dg, dg, wn, wl, we)


@jax.jit
def kernel(x, edge_index, edge_type, rel_emb, weight_neighbor, loop_weight,
           evolve_loop_weight):
    n, D = x.shape
    e = edge_index.shape[1]
    Dh = D // 2

    per_tile_chunks = -(-e // (NTILES * C))
    e_pad = NTILES * per_tile_chunks * C
    n_pad = -(-(n + 1) // (8 * SUB)) * (8 * SUB)

    xa = x[:, :Dh]
    xb = x[:, Dh:]
    ra = rel_emb[:, :Dh]
    rb = rel_emb[:, Dh:]

    src = jnp.concatenate(
        [edge_index[0], jnp.zeros((e_pad - e,), jnp.int32)])
    dst = jnp.concatenate(
        [edge_index[1], jnp.full((e_pad - e,), n, jnp.int32)])
    et = jnp.concatenate(
        [edge_type, jnp.zeros((e_pad - e,), jnp.int32)])
    shp = (2, SUB, per_tile_chunks, C)
    src4 = src.reshape(shp)
    dst4 = dst.reshape(shp)
    et4 = et.reshape(shp)

    sa, dg = _sc_pass(xa, ra, src4, dst4, et4, n_pad, per_tile_chunks,
                      with_deg=True)
    sb, = _sc_pass(xb, rb, src4, dst4, et4, n_pad, per_tile_chunks,
                   with_deg=False)

    return _tc_combine(
        x, sa, sb, dg,
        weight_neighbor, loop_weight, evolve_loop_weight, blk=1000)
